# baseline (speedup 1.0000x reference)
import functools

import jax
import jax.numpy as jnp
from jax import lax
from jax.experimental import pallas as pl
from jax.experimental.pallas import tpu as pltpu
from jax.experimental.pallas import tpu_sc as plsc

B, N, D, DFF, V = 256, 512, 256, 512, 100000

SC_NC, SC_NS = 2, 16
SC_NW = SC_NC * SC_NS


def _sc_gather_rows(table, idx):
    n, d = idx.shape[0], table.shape[1]
    b_per_w = n // SC_NW
    mesh = plsc.VectorSubcoreMesh(core_axis_name="c", subcore_axis_name="s")

    @functools.partial(
        pl.kernel,
        out_type=jax.ShapeDtypeStruct((n, d), jnp.float32),
        mesh=mesh,
        scratch_types=[
            pltpu.VMEM((b_per_w,), jnp.int32),
            pltpu.VMEM((b_per_w, d), jnp.float32),
            pltpu.SemaphoreType.DMA,
        ],
    )
    def gather_kernel(table_hbm, idx_hbm, out_hbm, idx_v, rows_v, sem):
        wid = lax.axis_index("s") * SC_NC + lax.axis_index("c")
        base = wid * b_per_w
        pltpu.sync_copy(idx_hbm.at[pl.ds(base, b_per_w)], idx_v)
        pltpu.async_copy(table_hbm.at[idx_v], rows_v, sem).wait()
        pltpu.sync_copy(rows_v, out_hbm.at[pl.ds(base, b_per_w)])

    return gather_kernel(table, idx)


def _masked_sum_kernel(maskf_ref, eh_ref, out_ref):
    i = pl.program_id(0)

    @pl.when(i == 0)
    def _():
        out_ref[...] = jnp.zeros_like(out_ref)

    out_ref[...] += jnp.dot(
        maskf_ref[...], eh_ref[...], preferred_element_type=jnp.float32, precision=jax.lax.Precision.HIGHEST
    )


def _masked_sum(maskf_row, eh2d, block_rows, interpret=False):
    total = eh2d.shape[0]
    nb = total // block_rows
    return pl.pallas_call(
        _masked_sum_kernel,
        grid=(nb,),
        in_specs=[
            pl.BlockSpec((1, block_rows), lambda i: (0, i)),
            pl.BlockSpec((block_rows, D), lambda i: (i, 0)),
        ],
        out_specs=pl.BlockSpec((1, D), lambda i: (0, 0)),
        out_shape=jax.ShapeDtypeStruct((1, D), jnp.float32),
        interpret=interpret,
    )(maskf_row, eh2d)


def _bdot(a, b):
    return jnp.dot(a.astype(jnp.bfloat16), b.astype(jnp.bfloat16),
                   preferred_element_type=jnp.float32)


def _mlp_kernel(eq0_ref, wf1_ref, bf1_ref, wf2_ref, bf2_ref, wa_ref, h_ref,
                s_ref, x_ref):
    eq1 = jnp.maximum(_bdot(eq0_ref[...], wf1_ref[...]) + bf1_ref[...], 0.0)
    q = _bdot(eq1, wf2_ref[...]) + bf2_ref[...]
    qw = _bdot(q, wa_ref[...])
    u = qw * s_ref[...]
    x_ref[...] = q + _bdot(h_ref[...], u)


def _mlp(eq0, w_fc1, b_fc1, w_fc2, b_fc2, w_attn, h, s_row, interpret=False):
    return pl.pallas_call(
        _mlp_kernel,
        out_shape=jax.ShapeDtypeStruct((B, D), jnp.float32),
        interpret=interpret,
    )(eq0, w_fc1, b_fc1.reshape(1, DFF), w_fc2, b_fc2.reshape(1, D), w_attn,
      h, s_row)


def _softmax_kernel(x_ref, w_ref, b_ref, out_ref, m_ref, s_ref, *, tv, nt):
    p = pl.program_id(0)
    j = pl.program_id(1)

    logits = _bdot(x_ref[...], w_ref[...]) + b_ref[...]

    @pl.when(p == 0)
    def _pass_stats():
        @pl.when(j == 0)
        def _():
            m_ref[...] = jnp.full_like(m_ref, -1e30)
            s_ref[...] = jnp.zeros_like(s_ref)

        col = j * tv + lax.broadcasted_iota(jnp.int32, (B, tv), 1)
        lg = jnp.where(col < V, logits, -1e30)
        tile_max = jnp.max(lg, axis=1, keepdims=True)
        m_old = m_ref[...]
        m_new = jnp.maximum(m_old, tile_max)
        sum_exp = jnp.sum(jnp.exp(lg - m_new[:, :1]), axis=1, keepdims=True)
        s_ref[...] = s_ref[...] * jnp.exp(m_old - m_new) + sum_exp
        m_ref[...] = m_new

    @pl.when(p == 1)
    def _pass_write():
        m = m_ref[:, :1]
        rs = 1.0 / s_ref[:, :1]
        out_ref[...] = jnp.exp(logits - m) * rs


def _softmax_matmul(x, w_sm, b_sm_row, tv, interpret=False):
    nt = pl.cdiv(V, tv)
    kern = functools.partial(_softmax_kernel, tv=tv, nt=nt)
    return pl.pallas_call(
        kern,
        grid=(2, nt),
        in_specs=[
            pl.BlockSpec((B, D), lambda p, j: (0, 0)),
            pl.BlockSpec((D, tv), lambda p, j: (0, j)),
            pl.BlockSpec((1, tv), lambda p, j: (0, j)),
        ],
        out_specs=pl.BlockSpec((B, tv), lambda p, j: (0, jnp.where(p == 0, 0, j))),
        out_shape=jax.ShapeDtypeStruct((B, V), jnp.float32),
        scratch_shapes=[
            pltpu.VMEM((B, 128), jnp.float32),
            pltpu.VMEM((B, 128), jnp.float32),
        ],
        interpret=interpret,
    )(x, w_sm, b_sm_row)


def kernel(entity_hiddens, question, keys_mask, W_emb, W_fc1, b_fc1, W_fc2,
           b_fc2, W_attn, H, W_sm, b_sm):
    eq0 = _sc_gather_rows(W_emb, question)
    maskf_row = keys_mask.astype(jnp.float32).reshape(1, B * N)
    eh2d = entity_hiddens.reshape(B * N, D)
    s_row = _masked_sum(maskf_row, eh2d, block_rows=8192)
    x = _mlp(eq0, W_fc1, b_fc1, W_fc2, b_fc2, W_attn, H, s_row)
    return _softmax_matmul(x, W_sm, b_sm.reshape(1, V), tv=2048)
